# Initial kernel scaffold; baseline (speedup 1.0000x reference)
import jax
import jax.numpy as jnp
from jax.experimental import pallas as pl


def kernel(z, embeddings):
    raise NotImplementedError("write your pallas kernel here")


# speedup vs baseline: 1.3440x; 1.3440x over previous
import functools

import jax
import jax.numpy as jnp
from jax import lax
from jax.experimental import pallas as pl
from jax.experimental.pallas import tpu as pltpu
from jax.experimental.pallas import tpu_sc as plsc

N = 65536
D = 32
K = 8192
TILE = 512
GRID = N // TILE
BETA = 0.25

_NC = 2
_NS = 16
NW = _NC * _NS
BPW = N // NW
L = 16


def _argmin_body(z_ref, e_ref, idx_ref, dsum_ref, cnt_ref):
    i = pl.program_id(0)
    z = z_ref[...]
    e = e_ref[...]
    z2 = jnp.sum(z * z, axis=1, keepdims=True)
    e2 = jnp.sum(e * e, axis=0, keepdims=True)
    c = lax.dot_general(z, e, (((1,), (0,)), ((), ())),
                        preferred_element_type=jnp.float32)
    s = (z2 + e2) - 2.0 * c
    H = K // 2
    sl = s[:, :H]
    sr = s[:, H:]
    mvl = jnp.min(sl, axis=1, keepdims=True)
    mvr = jnp.min(sr, axis=1, keepdims=True)
    iota_h = lax.broadcasted_iota(jnp.int32, sl.shape, 1)
    il = jnp.min(jnp.where(sl == mvl, iota_h, H), axis=1)
    ir = jnp.min(jnp.where(sr == mvr, iota_h, H), axis=1) + H
    mvl_bf = mvl.astype(jnp.bfloat16).astype(jnp.float32)
    take_r = mvr < mvl_bf
    idx = jnp.where(take_r[:, 0], ir, il)
    mv = jnp.where(take_r, mvr, mvl)
    idx_ref[0, 0, :] = idx
    part = jnp.sum(mv)
    iota = lax.broadcasted_iota(jnp.int32, s.shape, 1)
    onehot = (iota == idx[:, None]).astype(jnp.float32)
    cnt_part = jnp.sum(onehot, axis=0, keepdims=True)

    @pl.when(i == 0)
    def _():
        dsum_ref[...] = jnp.zeros_like(dsum_ref)
        cnt_ref[...] = jnp.zeros_like(cnt_ref)

    dsum_ref[...] = dsum_ref[...] + part
    cnt_ref[...] = cnt_ref[...] + cnt_part


_argmin_call = pl.pallas_call(
    _argmin_body,
    grid=(GRID,),
    in_specs=[
        pl.BlockSpec((TILE, D), lambda i: (i, 0)),
        pl.BlockSpec((D, K), lambda i: (0, 0)),
    ],
    out_specs=[
        pl.BlockSpec((1, 1, TILE), lambda i: (i, 0, 0)),
        pl.BlockSpec((1, 128), lambda i: (0, 0)),
        pl.BlockSpec((1, K), lambda i: (0, 0)),
    ],
    out_shape=[
        jax.ShapeDtypeStruct((GRID, 1, TILE), jnp.int32),
        jax.ShapeDtypeStruct((1, 128), jnp.float32),
        jax.ShapeDtypeStruct((1, K), jnp.float32),
    ],
)


@functools.cache
def _gather_call():

    @functools.partial(
        pl.kernel,
        out_type=jax.ShapeDtypeStruct((N, D), jnp.float32),
        mesh=plsc.VectorSubcoreMesh(core_axis_name="c", subcore_axis_name="s",
                                    num_cores=_NC, num_subcores=_NS),
        scratch_types=[
            pltpu.VMEM((BPW,), jnp.int32),
            pltpu.VMEM((BPW, D), jnp.float32),
            pltpu.SemaphoreType.DMA,
        ],
        compiler_params=pltpu.CompilerParams(use_tc_tiling_on_sc=False),
    )
    def _gather(idx_hbm, table_hbm, q_hbm, idx_v, rows_v, sem):
        wid = lax.axis_index("s") * _NC + lax.axis_index("c")
        base = wid * BPW
        pltpu.sync_copy(idx_hbm.at[pl.ds(base, BPW)], idx_v)
        pltpu.async_copy(table_hbm.at[idx_v], rows_v, sem).wait()
        pltpu.sync_copy(rows_v, q_hbm.at[pl.ds(base, BPW)])

    return _gather


def _scalars_body(counts_ref, dsum_ref, vq_ref, com_ref, perp_ref):
    p = counts_ref[...] * (1.0 / N)
    ent = p * jnp.log(p + 1e-10)
    perp_ref[0, 0] = jnp.exp(-jnp.sum(ent))
    vq = dsum_ref[0, 0] * (1.0 / (N * D))
    vq_ref[0, 0] = vq
    com_ref[0, 0] = BETA * vq


_scalars_call = pl.pallas_call(
    _scalars_body,
    in_specs=[
        pl.BlockSpec((1, K), lambda: (0, 0)),
        pl.BlockSpec(memory_space=pltpu.SMEM),
    ],
    out_specs=[
        pl.BlockSpec(memory_space=pltpu.SMEM),
        pl.BlockSpec(memory_space=pltpu.SMEM),
        pl.BlockSpec(memory_space=pltpu.SMEM),
    ],
    out_shape=[
        jax.ShapeDtypeStruct((1, 1), jnp.float32),
        jax.ShapeDtypeStruct((1, 1), jnp.float32),
        jax.ShapeDtypeStruct((1, 1), jnp.float32),
    ],
)


def kernel(z, embeddings):
    z_flat = z.reshape(N, D)
    idx3, dsum, counts = _argmin_call(z_flat, embeddings)
    idx = idx3.reshape(N)
    table = embeddings.T
    q = _gather_call()(idx, table)
    vq, com, perp = _scalars_call(counts, dsum)
    quantized_st = q.reshape(z.shape)
    return (quantized_st, idx, vq.reshape(()), com.reshape(()),
            perp.reshape(()))


# speedup vs baseline: 1.7936x; 1.3345x over previous
import functools

import jax
import jax.numpy as jnp
from jax import lax
from jax.experimental import pallas as pl
from jax.experimental.pallas import tpu as pltpu
from jax.experimental.pallas import tpu_sc as plsc

N = 65536
D = 32
K = 8192
TILE = 512
GRID = N // TILE
BETA = 0.25

_NC = 2
_NS = 16
NW = _NC * _NS
BPW = N // NW
L = 16


def _argmin_body(z_ref, e_ref, idx_ref, dsum_ref, cnt_ref):
    i = pl.program_id(0)
    z = z_ref[...]
    e = e_ref[...]
    z2 = jnp.sum(z * z, axis=1, keepdims=True)
    e2 = jnp.sum(e * e, axis=0, keepdims=True)
    c2 = lax.dot_general(z, e * (-2.0), (((1,), (0,)), ((), ())),
                         preferred_element_type=jnp.float32)
    s = (z2 + e2) + c2
    H = K // 2
    sl = s[:, :H]
    sr = s[:, H:]
    mvl = jnp.min(sl, axis=1, keepdims=True)
    mvr = jnp.min(sr, axis=1, keepdims=True)
    iota_f = lax.broadcasted_iota(jnp.int32, sl.shape, 1).astype(jnp.float32)
    il = jnp.min(jnp.where(sl == mvl, iota_f, float(H)), axis=1)
    ir = jnp.min(jnp.where(sr == mvr, iota_f, float(H)), axis=1) + float(H)
    mvl_bf = mvl.astype(jnp.bfloat16).astype(jnp.float32)
    take_r = mvr < mvl_bf
    idx = jnp.where(take_r[:, 0], ir, il).astype(jnp.int32)
    mv = jnp.where(take_r, mvr, mvl)
    idx_ref[0, 0, :] = idx
    part = jnp.sum(mv)
    hi = idx[:, None] // 128
    lo = idx[:, None] % 128
    oh_hi = (lax.broadcasted_iota(jnp.int32, (TILE, K // 128), 1)
             == hi).astype(jnp.float32)
    oh_lo = (lax.broadcasted_iota(jnp.int32, (TILE, 128), 1)
             == lo).astype(jnp.float32)
    cnt_part = lax.dot_general(oh_hi, oh_lo, (((0,), (0,)), ((), ())),
                               preferred_element_type=jnp.float32)

    @pl.when(i == 0)
    def _():
        dsum_ref[...] = jnp.zeros_like(dsum_ref)
        cnt_ref[...] = jnp.zeros_like(cnt_ref)

    dsum_ref[...] = dsum_ref[...] + part
    cnt_ref[...] = cnt_ref[...] + cnt_part


_argmin_call = pl.pallas_call(
    _argmin_body,
    grid=(GRID,),
    in_specs=[
        pl.BlockSpec((TILE, D), lambda i: (i, 0)),
        pl.BlockSpec((D, K), lambda i: (0, 0)),
    ],
    out_specs=[
        pl.BlockSpec((1, 1, TILE), lambda i: (i, 0, 0)),
        pl.BlockSpec((1, 128), lambda i: (0, 0)),
        pl.BlockSpec((K // 128, 128), lambda i: (0, 0)),
    ],
    out_shape=[
        jax.ShapeDtypeStruct((GRID, 1, TILE), jnp.int32),
        jax.ShapeDtypeStruct((1, 128), jnp.float32),
        jax.ShapeDtypeStruct((K // 128, 128), jnp.float32),
    ],
)


@functools.cache
def _gather_call():

    @functools.partial(
        pl.kernel,
        out_type=jax.ShapeDtypeStruct((N, D), jnp.float32),
        mesh=plsc.VectorSubcoreMesh(core_axis_name="c", subcore_axis_name="s",
                                    num_cores=_NC, num_subcores=_NS),
        scratch_types=[
            pltpu.VMEM((BPW,), jnp.int32),
            pltpu.VMEM((BPW, D), jnp.float32),
            pltpu.SemaphoreType.DMA,
        ],
        compiler_params=pltpu.CompilerParams(use_tc_tiling_on_sc=False),
    )
    def _gather(idx_hbm, table_hbm, q_hbm, idx_v, rows_v, sem):
        wid = lax.axis_index("s") * _NC + lax.axis_index("c")
        base = wid * BPW
        pltpu.sync_copy(idx_hbm.at[pl.ds(base, BPW)], idx_v)
        pltpu.async_copy(table_hbm.at[idx_v], rows_v, sem).wait()
        pltpu.sync_copy(rows_v, q_hbm.at[pl.ds(base, BPW)])

    return _gather


def _scalars_body(counts_ref, dsum_ref, vq_ref, com_ref, perp_ref):
    p = counts_ref[...] * (1.0 / N)
    ent = p * jnp.log(p + 1e-10)
    perp_ref[0, 0] = jnp.exp(-jnp.sum(ent))
    vq = dsum_ref[0, 0] * (1.0 / (N * D))
    vq_ref[0, 0] = vq
    com_ref[0, 0] = BETA * vq


_scalars_call = pl.pallas_call(
    _scalars_body,
    in_specs=[
        pl.BlockSpec((K // 128, 128), lambda: (0, 0)),
        pl.BlockSpec(memory_space=pltpu.SMEM),
    ],
    out_specs=[
        pl.BlockSpec(memory_space=pltpu.SMEM),
        pl.BlockSpec(memory_space=pltpu.SMEM),
        pl.BlockSpec(memory_space=pltpu.SMEM),
    ],
    out_shape=[
        jax.ShapeDtypeStruct((1, 1), jnp.float32),
        jax.ShapeDtypeStruct((1, 1), jnp.float32),
        jax.ShapeDtypeStruct((1, 1), jnp.float32),
    ],
)


def kernel(z, embeddings):
    z_flat = z.reshape(N, D)
    idx3, dsum, counts = _argmin_call(z_flat, embeddings)
    idx = idx3.reshape(N)
    table = embeddings.T
    q = _gather_call()(idx, table)
    vq, com, perp = _scalars_call(counts, dsum)
    quantized_st = q.reshape(z.shape)
    return (quantized_st, idx, vq.reshape(()), com.reshape(()),
            perp.reshape(()))


# speedup vs baseline: 1.9079x; 1.0637x over previous
import functools

import jax
import jax.numpy as jnp
from jax import lax
from jax.experimental import pallas as pl
from jax.experimental.pallas import tpu as pltpu
from jax.experimental.pallas import tpu_sc as plsc

N = 65536
D = 32
K = 8192
TILE = 1024
GRID = N // TILE
BETA = 0.25

_NC = 2
_NS = 16
NW = _NC * _NS
BPW = N // NW
L = 16


def _argmin_body(z_ref, e_ref, idx_ref, dsum_ref, cnt_ref):
    i = pl.program_id(0)
    z = z_ref[...]
    e = e_ref[...]
    z2 = jnp.sum(z * z, axis=1, keepdims=True)
    e2 = jnp.sum(e * e, axis=0, keepdims=True)
    c2 = lax.dot_general(z, e * (-2.0), (((1,), (0,)), ((), ())),
                         preferred_element_type=jnp.float32)
    s = (z2 + e2) + c2
    H = K // 2
    sl = s[:, :H]
    sr = s[:, H:]
    mvl = jnp.min(sl, axis=1, keepdims=True)
    mvr = jnp.min(sr, axis=1, keepdims=True)
    iota_f = lax.broadcasted_iota(jnp.int32, sl.shape, 1).astype(jnp.float32)
    il = jnp.min(jnp.where(sl == mvl, iota_f, float(H)), axis=1)
    ir = jnp.min(jnp.where(sr == mvr, iota_f, float(H)), axis=1) + float(H)
    mvl_bf = mvl.astype(jnp.bfloat16).astype(jnp.float32)
    take_r = mvr < mvl_bf
    idx = jnp.where(take_r[:, 0], ir, il).astype(jnp.int32)
    mv = jnp.where(take_r, mvr, mvl)
    idx_ref[0, 0, :] = idx
    part = jnp.sum(mv)
    hi = idx[:, None] // 128
    lo = idx[:, None] % 128
    oh_hi = (lax.broadcasted_iota(jnp.int32, (TILE, K // 128), 1)
             == hi).astype(jnp.float32)
    oh_lo = (lax.broadcasted_iota(jnp.int32, (TILE, 128), 1)
             == lo).astype(jnp.float32)
    cnt_part = lax.dot_general(oh_hi, oh_lo, (((0,), (0,)), ((), ())),
                               preferred_element_type=jnp.float32)

    @pl.when(i == 0)
    def _():
        dsum_ref[...] = jnp.zeros_like(dsum_ref)
        cnt_ref[...] = jnp.zeros_like(cnt_ref)

    dsum_ref[...] = dsum_ref[...] + part
    cnt_ref[...] = cnt_ref[...] + cnt_part


_argmin_call = pl.pallas_call(
    _argmin_body,
    grid=(GRID,),
    in_specs=[
        pl.BlockSpec((TILE, D), lambda i: (i, 0)),
        pl.BlockSpec((D, K), lambda i: (0, 0)),
    ],
    out_specs=[
        pl.BlockSpec((1, 1, TILE), lambda i: (i, 0, 0)),
        pl.BlockSpec((1, 128), lambda i: (0, 0)),
        pl.BlockSpec((K // 128, 128), lambda i: (0, 0)),
    ],
    out_shape=[
        jax.ShapeDtypeStruct((GRID, 1, TILE), jnp.int32),
        jax.ShapeDtypeStruct((1, 128), jnp.float32),
        jax.ShapeDtypeStruct((K // 128, 128), jnp.float32),
    ],
)


@functools.cache
def _gather_call():

    @functools.partial(
        pl.kernel,
        out_type=jax.ShapeDtypeStruct((N, D), jnp.float32),
        mesh=plsc.VectorSubcoreMesh(core_axis_name="c", subcore_axis_name="s",
                                    num_cores=_NC, num_subcores=_NS),
        scratch_types=[
            pltpu.VMEM((BPW,), jnp.int32),
            pltpu.VMEM((BPW, D), jnp.float32),
            pltpu.SemaphoreType.DMA,
        ],
        compiler_params=pltpu.CompilerParams(use_tc_tiling_on_sc=False),
    )
    def _gather(idx_hbm, table_hbm, q_hbm, idx_v, rows_v, sem):
        wid = lax.axis_index("s") * _NC + lax.axis_index("c")
        base = wid * BPW
        pltpu.sync_copy(idx_hbm.at[pl.ds(base, BPW)], idx_v)
        pltpu.async_copy(table_hbm.at[idx_v], rows_v, sem).wait()
        pltpu.sync_copy(rows_v, q_hbm.at[pl.ds(base, BPW)])

    return _gather


def _scalars_body(counts_ref, dsum_ref, vq_ref, com_ref, perp_ref):
    p = counts_ref[...] * (1.0 / N)
    ent = p * jnp.log(p + 1e-10)
    perp_ref[0, 0] = jnp.exp(-jnp.sum(ent))
    vq = dsum_ref[0, 0] * (1.0 / (N * D))
    vq_ref[0, 0] = vq
    com_ref[0, 0] = BETA * vq


_scalars_call = pl.pallas_call(
    _scalars_body,
    in_specs=[
        pl.BlockSpec((K // 128, 128), lambda: (0, 0)),
        pl.BlockSpec(memory_space=pltpu.SMEM),
    ],
    out_specs=[
        pl.BlockSpec(memory_space=pltpu.SMEM),
        pl.BlockSpec(memory_space=pltpu.SMEM),
        pl.BlockSpec(memory_space=pltpu.SMEM),
    ],
    out_shape=[
        jax.ShapeDtypeStruct((1, 1), jnp.float32),
        jax.ShapeDtypeStruct((1, 1), jnp.float32),
        jax.ShapeDtypeStruct((1, 1), jnp.float32),
    ],
)


def kernel(z, embeddings):
    z_flat = z.reshape(N, D)
    idx3, dsum, counts = _argmin_call(z_flat, embeddings)
    idx = idx3.reshape(N)
    table = embeddings.T
    q = _gather_call()(idx, table)
    vq, com, perp = _scalars_call(counts, dsum)
    quantized_st = q.reshape(z.shape)
    return (quantized_st, idx, vq.reshape(()), com.reshape(()),
            perp.reshape(()))


# speedup vs baseline: 1.9105x; 1.0013x over previous
import functools

import jax
import jax.numpy as jnp
from jax import lax
from jax.experimental import pallas as pl
from jax.experimental.pallas import tpu as pltpu
from jax.experimental.pallas import tpu_sc as plsc

N = 65536
D = 32
K = 8192
TILE = 1024
GRID = N // TILE
BETA = 0.25

_NC = 2
_NS = 16
NW = _NC * _NS
BPW = N // NW
L = 16


def _argmin_body(z_ref, e_ref, idx_ref, vq_ref, com_ref, perp_ref,
                 dsum_ref, cnt_ref):
    i = pl.program_id(0)
    z = z_ref[...]
    e = e_ref[...]
    z2 = jnp.sum(z * z, axis=1, keepdims=True)
    e2 = jnp.sum(e * e, axis=0, keepdims=True)
    c2 = lax.dot_general(z, e * (-2.0), (((1,), (0,)), ((), ())),
                         preferred_element_type=jnp.float32)
    s = (z2 + e2) + c2
    H = K // 2
    sl = s[:, :H]
    sr = s[:, H:]
    mvl = jnp.min(sl, axis=1, keepdims=True)
    mvr = jnp.min(sr, axis=1, keepdims=True)
    iota_f = lax.broadcasted_iota(jnp.int32, sl.shape, 1).astype(jnp.float32)
    il = jnp.min(jnp.where(sl == mvl, iota_f, float(H)), axis=1)
    ir = jnp.min(jnp.where(sr == mvr, iota_f, float(H)), axis=1) + float(H)
    mvl_bf = mvl.astype(jnp.bfloat16).astype(jnp.float32)
    take_r = mvr < mvl_bf
    idx = jnp.where(take_r[:, 0], ir, il).astype(jnp.int32)
    mv = jnp.where(take_r, mvr, mvl)
    idx_ref[0, 0, :] = idx
    part = jnp.sum(mv)
    hi = idx[:, None] // 128
    lo = idx[:, None] % 128
    oh_hi = (lax.broadcasted_iota(jnp.int32, (TILE, K // 128), 1)
             == hi).astype(jnp.float32)
    oh_lo = (lax.broadcasted_iota(jnp.int32, (TILE, 128), 1)
             == lo).astype(jnp.float32)
    cnt_part = lax.dot_general(oh_hi, oh_lo, (((0,), (0,)), ((), ())),
                               preferred_element_type=jnp.float32)

    @pl.when(i == 0)
    def _():
        dsum_ref[...] = jnp.zeros_like(dsum_ref)
        cnt_ref[...] = jnp.zeros_like(cnt_ref)

    dsum_ref[...] = dsum_ref[...] + part
    cnt_ref[...] = cnt_ref[...] + cnt_part

    @pl.when(i == GRID - 1)
    def _():
        p = cnt_ref[...] * (1.0 / N)
        ent = p * jnp.log(p + 1e-10)
        perp_ref[0, 0] = jnp.exp(-jnp.sum(ent))
        vq = dsum_ref[0, 0] * (1.0 / (N * D))
        vq_ref[0, 0] = vq
        com_ref[0, 0] = BETA * vq


_argmin_call = pl.pallas_call(
    _argmin_body,
    grid=(GRID,),
    in_specs=[
        pl.BlockSpec((TILE, D), lambda i: (i, 0)),
        pl.BlockSpec((D, K), lambda i: (0, 0)),
    ],
    out_specs=[
        pl.BlockSpec((1, 1, TILE), lambda i: (i, 0, 0)),
        pl.BlockSpec(memory_space=pltpu.SMEM),
        pl.BlockSpec(memory_space=pltpu.SMEM),
        pl.BlockSpec(memory_space=pltpu.SMEM),
    ],
    out_shape=[
        jax.ShapeDtypeStruct((GRID, 1, TILE), jnp.int32),
        jax.ShapeDtypeStruct((1, 1), jnp.float32),
        jax.ShapeDtypeStruct((1, 1), jnp.float32),
        jax.ShapeDtypeStruct((1, 1), jnp.float32),
    ],
    scratch_shapes=[
        pltpu.VMEM((1, 128), jnp.float32),
        pltpu.VMEM((K // 128, 128), jnp.float32),
    ],
)


@functools.cache
def _gather_call():

    @functools.partial(
        pl.kernel,
        out_type=jax.ShapeDtypeStruct((N, D), jnp.float32),
        mesh=plsc.VectorSubcoreMesh(core_axis_name="c", subcore_axis_name="s",
                                    num_cores=_NC, num_subcores=_NS),
        scratch_types=[
            pltpu.VMEM((BPW,), jnp.int32),
            pltpu.VMEM((BPW, D), jnp.float32),
            pltpu.SemaphoreType.DMA,
        ],
        compiler_params=pltpu.CompilerParams(use_tc_tiling_on_sc=False),
    )
    def _gather(idx_hbm, table_hbm, q_hbm, idx_v, rows_v, sem):
        wid = lax.axis_index("s") * _NC + lax.axis_index("c")
        base = wid * BPW
        pltpu.sync_copy(idx_hbm.at[pl.ds(base, BPW)], idx_v)
        pltpu.async_copy(table_hbm.at[idx_v], rows_v, sem).wait()
        pltpu.sync_copy(rows_v, q_hbm.at[pl.ds(base, BPW)])

    return _gather


def kernel(z, embeddings):
    z_flat = z.reshape(N, D)
    idx3, vq, com, perp = _argmin_call(z_flat, embeddings)
    idx = idx3.reshape(N)
    table = embeddings.T
    q = _gather_call()(idx, table)
    quantized_st = q.reshape(z.shape)
    return (quantized_st, idx, vq.reshape(()), com.reshape(()),
            perp.reshape(()))
